# baseline (speedup 1.0000x reference)
import dataclasses
import functools

import jax
import jax.numpy as jnp
from jax import lax
from jax.experimental import pallas as pl
from jax.experimental.pallas import tpu as pltpu
from jax.experimental.pallas import tpu_sc as plsc

N = 10000
E = 320000
D = 128
H = 128

C = 128
R = E // C
NP = 10240
TILE_ROWS = NP // 16

_mesh = plsc.VectorSubcoreMesh(core_axis_name="c", subcore_axis_name="s")

_sc_params = pltpu.CompilerParams()
if "needs_layout_passes" in pltpu.CompilerParams.__dataclass_fields__:
    _sc_params = dataclasses.replace(_sc_params, needs_layout_passes=False)


@functools.partial(
    pl.kernel,
    out_type=[
        jax.ShapeDtypeStruct((2, NP, D), jnp.float32),
        jax.ShapeDtypeStruct((2, NP), jnp.float32),
    ],
    mesh=_mesh,
    scratch_types=[
        pltpu.VMEM((C,), jnp.int32),
        pltpu.VMEM((C,), jnp.int32),
        pltpu.VMEM((C,), jnp.int32),
        pltpu.VMEM((C,), jnp.int32),
        pltpu.VMEM((C, D), jnp.float32),
        pltpu.VMEM((C, D), jnp.float32),
        pltpu.VMEM((C,), jnp.float32),
        pltpu.VMEM((TILE_ROWS,), jnp.float32),
        pltpu.VMEM_SHARED((NP, D), jnp.float32),
        pltpu.VMEM_SHARED((NP,), jnp.float32),
        pltpu.SemaphoreType.DMA,
        pltpu.SemaphoreType.DMA,
        pltpu.SemaphoreType.DMA,
        pltpu.SemaphoreType.DMA,
        pltpu.SemaphoreType.DMA,
        pltpu.SemaphoreType.DMA,
    ],
)
def _sc_seg_sum(x_hbm, src_hbm, dst_hbm, sums_hbm, cnts_hbm,
                srcv0, dstv0, srcv1, dstv1, rows, rows1, ones, zcnt,
                acc_sh, cnt_sh, sem, sem1, ss0, ss1, cs0, cs1):
    cid = lax.axis_index("c")
    sid = lax.axis_index("s")
    srcv = (srcv0, srcv1)
    dstv = (dstv0, dstv1)
    rowsb = (rows, rows1)
    gsem = (sem, sem1)
    ssem = (ss0, ss1)
    csem = (cs0, cs1)

    @pl.loop(0, C, step=16)
    def _(j):
        ones[pl.ds(j, 16)] = jnp.ones((16,), jnp.float32)

    @pl.loop(0, C)
    def _(i):
        @pl.loop(0, D, step=16)
        def _(j):
            rows[i, pl.ds(j, 16)] = jnp.zeros((16,), jnp.float32)

    @pl.loop(0, TILE_ROWS, step=16)
    def _(j):
        zcnt[pl.ds(j, 16)] = jnp.zeros((16,), jnp.float32)

    base = sid * TILE_ROWS
    for k in range(TILE_ROWS // C):
        pltpu.sync_copy(rows, acc_sh.at[pl.ds(base + k * C, C)])
    pltpu.sync_copy(zcnt, cnt_sh.at[pl.ds(base, TILE_ROWS)])
    plsc.subcore_barrier()

    lo = cid * (R // 2) + sid
    hi = (cid + 1) * (R // 2)

    pltpu.sync_copy(src_hbm.at[lo], srcv0)
    pltpu.sync_copy(dst_hbm.at[lo], dstv0)
    pltpu.async_copy(x_hbm.at[srcv0], rows, sem)

    @pl.loop(lo, hi, step=32)
    def _(r0):
        for b in range(2):
            r = r0 + b * 16
            p, q = b, 1 - b

            @pl.when(r < hi)
            def _():
                pltpu.make_async_copy(x_hbm.at[pl.ds(0, C)], rowsb[p],
                                      gsem[p]).wait()
                pltpu.async_copy(rowsb[p], acc_sh.at[dstv[p]], ssem[p],
                                 add=True)
                pltpu.async_copy(ones, cnt_sh.at[dstv[p]], csem[p],
                                 add=True)

                @pl.when(r + 16 < hi)
                def _():
                    @pl.when(r > lo)
                    def _():
                        pltpu.make_async_copy(rowsb[q],
                                              acc_sh.at[pl.ds(0, C)],
                                              ssem[q]).wait()
                        pltpu.make_async_copy(ones,
                                              cnt_sh.at[pl.ds(0, C)],
                                              csem[q]).wait()
                    pltpu.sync_copy(src_hbm.at[r + 16], srcv[q])
                    pltpu.sync_copy(dst_hbm.at[r + 16], dstv[q])
                    pltpu.async_copy(x_hbm.at[srcv[q]], rowsb[q],
                                     gsem[q])

    for p in range(2):
        pltpu.make_async_copy(rowsb[p], acc_sh.at[pl.ds(0, C)],
                              ssem[p]).wait()
        pltpu.make_async_copy(ones, cnt_sh.at[pl.ds(0, C)],
                              csem[p]).wait()

    plsc.subcore_barrier()

    pltpu.sync_copy(acc_sh.at[pl.ds(base, TILE_ROWS)],
                    sums_hbm.at[cid, pl.ds(base, TILE_ROWS)])
    pltpu.sync_copy(cnt_sh.at[pl.ds(base, TILE_ROWS)],
                    cnts_hbm.at[cid, pl.ds(base, TILE_ROWS)])


@functools.partial(
    pl.kernel,
    out_type=jax.ShapeDtypeStruct((2, NP), jnp.float32),
    mesh=_mesh,
    compiler_params=_sc_params,
    scratch_types=[
        pltpu.VMEM((C,), jnp.int32),
        pltpu.VMEM((C,), jnp.int32),
        pltpu.VMEM((C,), jnp.float32),
        pltpu.VMEM((NP,), jnp.float32),
        pltpu.VMEM((TILE_ROWS,), jnp.float32),
        pltpu.VMEM_SHARED((NP,), jnp.float32),
    ],
)
def _sc_seg_sum_scalar(s2_hbm, src_hbm, dst_hbm, parts_hbm,
                       srcv, dstv, vals, s2loc, zcnt, acc_sh):
    cid = lax.axis_index("c")
    sid = lax.axis_index("s")

    @pl.loop(0, TILE_ROWS, step=16)
    def _(j):
        zcnt[pl.ds(j, 16)] = jnp.zeros((16,), jnp.float32)

    base = sid * TILE_ROWS
    pltpu.sync_copy(zcnt, acc_sh.at[pl.ds(base, TILE_ROWS)])
    pltpu.sync_copy(s2_hbm, s2loc)
    plsc.subcore_barrier()

    lo = cid * (R // 2) + sid
    hi = (cid + 1) * (R // 2)

    @pl.loop(lo, hi, step=16)
    def _(r):
        pltpu.sync_copy(src_hbm.at[r], srcv)
        pltpu.sync_copy(dst_hbm.at[r], dstv)
        for j in range(C // 16):
            idx = srcv[pl.ds(j * 16, 16)]
            vals[pl.ds(j * 16, 16)] = plsc.load_gather(s2loc, [idx])
        pltpu.sync_copy(vals, acc_sh.at[dstv], add=True)

    plsc.subcore_barrier()
    pltpu.sync_copy(acc_sh.at[pl.ds(base, TILE_ROWS)],
                    parts_hbm.at[cid, pl.ds(base, TILE_ROWS)])


def _tc_layer1(s_ref, c_ref, x_ref, wl1_ref, bl1_ref, wr1_ref,
               wl2_ref, wr2_ref, bl2_ref, s2_ref, r2b_ref, cntc_ref):
    seg = s_ref[0] + s_ref[1]
    cnt = jnp.maximum(c_ref[0] + c_ref[1], 1.0)
    agg = seg / cnt
    h = agg @ wl1_ref[...] + bl1_ref[...] + x_ref[...] @ wr1_ref[...]
    h = jnp.maximum(h, 0.0)
    s2_ref[...] = h @ wl2_ref[...]
    r2b_ref[...] = h @ wr2_ref[...] + bl2_ref[...]
    cntc_ref[...] = cnt


def _tc_head(p_ref, cnt_ref, r2b_ref, noise_ref, wmu_ref, bmu_ref,
             wlv_ref, blv_ref, z_ref):
    xm = (p_ref[0] + p_ref[1]) / cnt_ref[...] + r2b_ref[...]
    xm = xm[:N]
    mu = xm * wmu_ref[0, 0] + bmu_ref[0, 0]
    lv = xm * wlv_ref[0, 0] + blv_ref[0, 0]
    z_ref[...] = mu + noise_ref[...] * jnp.exp(lv)


def kernel(x, edge_index, W_l1, b_l1, W_r1, W_l2, b_l2, W_r2,
           w_mu, b_mu, w_lv, b_lv, noise):
    src2 = edge_index[0].reshape(R, C)
    dst2 = edge_index[1].reshape(R, C)
    xp = jnp.pad(x, ((0, NP - N), (0, 0)))

    sums, cnts = _sc_seg_sum(xp, src2, dst2)

    B = 640
    s2, r2b, cntc = pl.pallas_call(
        _tc_layer1,
        grid=(NP // B,),
        in_specs=[
            pl.BlockSpec((2, B, D), lambda i: (0, i, 0)),
            pl.BlockSpec((2, B, 1), lambda i: (0, i, 0)),
            pl.BlockSpec((B, D), lambda i: (i, 0)),
            pl.BlockSpec((D, H), lambda i: (0, 0)),
            pl.BlockSpec((1, H), lambda i: (0, 0)),
            pl.BlockSpec((D, H), lambda i: (0, 0)),
            pl.BlockSpec((H, 1), lambda i: (0, 0)),
            pl.BlockSpec((H, 1), lambda i: (0, 0)),
            pl.BlockSpec((1, 1), lambda i: (0, 0)),
        ],
        out_specs=[
            pl.BlockSpec((B, 1), lambda i: (i, 0)),
            pl.BlockSpec((B, 1), lambda i: (i, 0)),
            pl.BlockSpec((B, 1), lambda i: (i, 0)),
        ],
        out_shape=[
            jax.ShapeDtypeStruct((NP, 1), jnp.float32),
            jax.ShapeDtypeStruct((NP, 1), jnp.float32),
            jax.ShapeDtypeStruct((NP, 1), jnp.float32),
        ],
    )(sums, cnts.reshape(2, NP, 1), xp, W_l1, b_l1.reshape(1, H), W_r1,
      W_l2, W_r2, b_l2.reshape(1, 1))

    parts2 = _sc_seg_sum_scalar(s2.reshape(NP), src2, dst2)

    z = pl.pallas_call(
        _tc_head,
        grid=(1,),
        in_specs=[
            pl.BlockSpec((2, NP, 1), lambda i: (0, 0, 0)),
            pl.BlockSpec((NP, 1), lambda i: (0, 0)),
            pl.BlockSpec((NP, 1), lambda i: (0, 0)),
            pl.BlockSpec((N, 1), lambda i: (0, 0)),
            pl.BlockSpec((1, 1), lambda i: (0, 0)),
            pl.BlockSpec((1, 1), lambda i: (0, 0)),
            pl.BlockSpec((1, 1), lambda i: (0, 0)),
            pl.BlockSpec((1, 1), lambda i: (0, 0)),
        ],
        out_specs=pl.BlockSpec((N, 1), lambda i: (0, 0)),
        out_shape=jax.ShapeDtypeStruct((N, 1), jnp.float32),
    )(parts2.reshape(2, NP, 1), cntc, r2b, noise,
      w_mu, b_mu.reshape(1, 1), w_lv, b_lv.reshape(1, 1))

    return z


# speedup vs baseline: 1.1317x; 1.1317x over previous
import dataclasses
import functools

import jax
import jax.numpy as jnp
from jax import lax
from jax.experimental import pallas as pl
from jax.experimental.pallas import tpu as pltpu
from jax.experimental.pallas import tpu_sc as plsc

N = 10000
E = 320000
D = 128
H = 128

C = 128
R = E // C
NP = 10240
TILE_ROWS = NP // 16

_mesh = plsc.VectorSubcoreMesh(core_axis_name="c", subcore_axis_name="s")

_sc_params = pltpu.CompilerParams()
if "needs_layout_passes" in pltpu.CompilerParams.__dataclass_fields__:
    _sc_params = dataclasses.replace(_sc_params, needs_layout_passes=False)


@functools.partial(
    pl.kernel,
    out_type=[
        jax.ShapeDtypeStruct((2, NP, D), jnp.float32),
        jax.ShapeDtypeStruct((2, NP), jnp.float32),
    ],
    mesh=_mesh,
    scratch_types=[
        pltpu.VMEM((C,), jnp.int32),
        pltpu.VMEM((C,), jnp.int32),
        pltpu.VMEM((C,), jnp.int32),
        pltpu.VMEM((C,), jnp.int32),
        pltpu.VMEM((C, D), jnp.float32),
        pltpu.VMEM((C, D), jnp.float32),
        pltpu.VMEM((C,), jnp.float32),
        pltpu.VMEM((TILE_ROWS,), jnp.float32),
        pltpu.VMEM_SHARED((NP, D), jnp.float32),
        pltpu.VMEM_SHARED((NP,), jnp.float32),
        pltpu.SemaphoreType.DMA,
        pltpu.SemaphoreType.DMA,
        pltpu.SemaphoreType.DMA,
        pltpu.SemaphoreType.DMA,
    ],
)
def _sc_seg_sum(x_hbm, src_hbm, dst_hbm, sums_hbm, cnts_hbm,
                srcv0, dstv0, srcv1, dstv1, rows, rows1, ones, zcnt,
                acc_sh, cnt_sh, sem, sem1, cs0, cs1):
    cid = lax.axis_index("c")
    sid = lax.axis_index("s")
    srcv = (srcv0, srcv1)
    dstv = (dstv0, dstv1)
    rowsb = (rows, rows1)
    gsem = (sem, sem1)
    csem = (cs0, cs1)

    @pl.loop(0, C, step=16)
    def _(j):
        ones[pl.ds(j, 16)] = jnp.ones((16,), jnp.float32)

    @pl.loop(0, C)
    def _(i):
        @pl.loop(0, D, step=16)
        def _(j):
            rows[i, pl.ds(j, 16)] = jnp.zeros((16,), jnp.float32)

    @pl.loop(0, TILE_ROWS, step=16)
    def _(j):
        zcnt[pl.ds(j, 16)] = jnp.zeros((16,), jnp.float32)

    base = sid * TILE_ROWS
    for k in range(TILE_ROWS // C):
        pltpu.sync_copy(rows, acc_sh.at[pl.ds(base + k * C, C)])
    pltpu.sync_copy(zcnt, cnt_sh.at[pl.ds(base, TILE_ROWS)])
    plsc.subcore_barrier()

    lo = cid * (R // 2) + sid
    hi = (cid + 1) * (R // 2)

    pltpu.sync_copy(src_hbm.at[lo], srcv0)
    pltpu.sync_copy(dst_hbm.at[lo], dstv0)
    pltpu.async_copy(x_hbm.at[srcv0], rows, sem)

    @pl.loop(lo, hi, step=32)
    def _(r0):
        for b in range(2):
            r = r0 + b * 16
            p, q = b, 1 - b

            @pl.when(r < hi)
            def _():
                @pl.when(r + 16 < hi)
                def _():
                    @pl.when(r > lo)
                    def _():
                        pltpu.make_async_copy(ones,
                                              cnt_sh.at[pl.ds(0, C)],
                                              csem[q]).wait()
                    pltpu.sync_copy(src_hbm.at[r + 16], srcv[q])
                    pltpu.sync_copy(dst_hbm.at[r + 16], dstv[q])
                    pltpu.async_copy(x_hbm.at[srcv[q]], rowsb[q],
                                     gsem[q])

                pltpu.make_async_copy(x_hbm.at[pl.ds(0, C)], rowsb[p],
                                      gsem[p]).wait()
                pltpu.sync_copy(rowsb[p], acc_sh.at[dstv[p]], add=True)
                pltpu.async_copy(ones, cnt_sh.at[dstv[p]], csem[p],
                                 add=True)

    for p in range(2):
        pltpu.make_async_copy(ones, cnt_sh.at[pl.ds(0, C)],
                              csem[p]).wait()

    plsc.subcore_barrier()

    pltpu.sync_copy(acc_sh.at[pl.ds(base, TILE_ROWS)],
                    sums_hbm.at[cid, pl.ds(base, TILE_ROWS)])
    pltpu.sync_copy(cnt_sh.at[pl.ds(base, TILE_ROWS)],
                    cnts_hbm.at[cid, pl.ds(base, TILE_ROWS)])


@functools.partial(
    pl.kernel,
    out_type=jax.ShapeDtypeStruct((2, NP), jnp.float32),
    mesh=_mesh,
    compiler_params=_sc_params,
    scratch_types=[
        pltpu.VMEM((C,), jnp.int32),
        pltpu.VMEM((C,), jnp.int32),
        pltpu.VMEM((C,), jnp.float32),
        pltpu.VMEM((NP,), jnp.float32),
        pltpu.VMEM((TILE_ROWS,), jnp.float32),
        pltpu.VMEM_SHARED((NP,), jnp.float32),
    ],
)
def _sc_seg_sum_scalar(s2_hbm, src_hbm, dst_hbm, parts_hbm,
                       srcv, dstv, vals, s2loc, zcnt, acc_sh):
    cid = lax.axis_index("c")
    sid = lax.axis_index("s")

    @pl.loop(0, TILE_ROWS, step=16)
    def _(j):
        zcnt[pl.ds(j, 16)] = jnp.zeros((16,), jnp.float32)

    base = sid * TILE_ROWS
    pltpu.sync_copy(zcnt, acc_sh.at[pl.ds(base, TILE_ROWS)])
    pltpu.sync_copy(s2_hbm, s2loc)
    plsc.subcore_barrier()

    lo = cid * (R // 2) + sid
    hi = (cid + 1) * (R // 2)

    @pl.loop(lo, hi, step=16)
    def _(r):
        pltpu.sync_copy(src_hbm.at[r], srcv)
        pltpu.sync_copy(dst_hbm.at[r], dstv)
        for j in range(C // 16):
            idx = srcv[pl.ds(j * 16, 16)]
            vals[pl.ds(j * 16, 16)] = plsc.load_gather(s2loc, [idx])
        pltpu.sync_copy(vals, acc_sh.at[dstv], add=True)

    plsc.subcore_barrier()
    pltpu.sync_copy(acc_sh.at[pl.ds(base, TILE_ROWS)],
                    parts_hbm.at[cid, pl.ds(base, TILE_ROWS)])


def _tc_layer1(s_ref, c_ref, x_ref, wl1_ref, bl1_ref, wr1_ref,
               wl2_ref, wr2_ref, bl2_ref, s2_ref, r2b_ref, cntc_ref):
    seg = s_ref[0] + s_ref[1]
    cnt = jnp.maximum(c_ref[0] + c_ref[1], 1.0)
    agg = seg / cnt
    h = agg @ wl1_ref[...] + bl1_ref[...] + x_ref[...] @ wr1_ref[...]
    h = jnp.maximum(h, 0.0)
    s2_ref[...] = h @ wl2_ref[...]
    r2b_ref[...] = h @ wr2_ref[...] + bl2_ref[...]
    cntc_ref[...] = cnt


def _tc_head(p_ref, cnt_ref, r2b_ref, noise_ref, wmu_ref, bmu_ref,
             wlv_ref, blv_ref, z_ref):
    xm = (p_ref[0] + p_ref[1]) / cnt_ref[...] + r2b_ref[...]
    xm = xm[:N]
    mu = xm * wmu_ref[0, 0] + bmu_ref[0, 0]
    lv = xm * wlv_ref[0, 0] + blv_ref[0, 0]
    z_ref[...] = mu + noise_ref[...] * jnp.exp(lv)


def kernel(x, edge_index, W_l1, b_l1, W_r1, W_l2, b_l2, W_r2,
           w_mu, b_mu, w_lv, b_lv, noise):
    src2 = edge_index[0].reshape(R, C)
    dst2 = edge_index[1].reshape(R, C)
    xp = jnp.pad(x, ((0, NP - N), (0, 0)))

    sums, cnts = _sc_seg_sum(xp, src2, dst2)

    B = 640
    s2, r2b, cntc = pl.pallas_call(
        _tc_layer1,
        grid=(NP // B,),
        in_specs=[
            pl.BlockSpec((2, B, D), lambda i: (0, i, 0)),
            pl.BlockSpec((2, B, 1), lambda i: (0, i, 0)),
            pl.BlockSpec((B, D), lambda i: (i, 0)),
            pl.BlockSpec((D, H), lambda i: (0, 0)),
            pl.BlockSpec((1, H), lambda i: (0, 0)),
            pl.BlockSpec((D, H), lambda i: (0, 0)),
            pl.BlockSpec((H, 1), lambda i: (0, 0)),
            pl.BlockSpec((H, 1), lambda i: (0, 0)),
            pl.BlockSpec((1, 1), lambda i: (0, 0)),
        ],
        out_specs=[
            pl.BlockSpec((B, 1), lambda i: (i, 0)),
            pl.BlockSpec((B, 1), lambda i: (i, 0)),
            pl.BlockSpec((B, 1), lambda i: (i, 0)),
        ],
        out_shape=[
            jax.ShapeDtypeStruct((NP, 1), jnp.float32),
            jax.ShapeDtypeStruct((NP, 1), jnp.float32),
            jax.ShapeDtypeStruct((NP, 1), jnp.float32),
        ],
    )(sums, cnts.reshape(2, NP, 1), xp, W_l1, b_l1.reshape(1, H), W_r1,
      W_l2, W_r2, b_l2.reshape(1, 1))

    parts2 = _sc_seg_sum_scalar(s2.reshape(NP), src2, dst2)

    z = pl.pallas_call(
        _tc_head,
        grid=(1,),
        in_specs=[
            pl.BlockSpec((2, NP, 1), lambda i: (0, 0, 0)),
            pl.BlockSpec((NP, 1), lambda i: (0, 0)),
            pl.BlockSpec((NP, 1), lambda i: (0, 0)),
            pl.BlockSpec((N, 1), lambda i: (0, 0)),
            pl.BlockSpec((1, 1), lambda i: (0, 0)),
            pl.BlockSpec((1, 1), lambda i: (0, 0)),
            pl.BlockSpec((1, 1), lambda i: (0, 0)),
            pl.BlockSpec((1, 1), lambda i: (0, 0)),
        ],
        out_specs=pl.BlockSpec((N, 1), lambda i: (0, 0)),
        out_shape=jax.ShapeDtypeStruct((N, 1), jnp.float32),
    )(parts2.reshape(2, NP, 1), cntc, r2b, noise,
      w_mu, b_mu.reshape(1, 1), w_lv, b_lv.reshape(1, 1))

    return z


# speedup vs baseline: 1.4470x; 1.2786x over previous
import dataclasses
import functools

import jax
import jax.numpy as jnp
from jax import lax
from jax.experimental import pallas as pl
from jax.experimental.pallas import tpu as pltpu
from jax.experimental.pallas import tpu_sc as plsc

N = 10000
E = 320000
D = 128
H = 128

C = 128
R = E // C
KPT = 80
RP = 32 * KPT
EP = RP * C
NP = 10240
TILE_ROWS = NP // 16

_mesh = plsc.VectorSubcoreMesh(core_axis_name="c", subcore_axis_name="s")

_sc_params = pltpu.CompilerParams()
if "needs_layout_passes" in pltpu.CompilerParams.__dataclass_fields__:
    _sc_params = dataclasses.replace(_sc_params, needs_layout_passes=False)


@functools.partial(
    pl.kernel,
    out_type=[
        jax.ShapeDtypeStruct((2, NP, D), jnp.float32),
        jax.ShapeDtypeStruct((2, NP), jnp.float32),
    ],
    mesh=_mesh,
    scratch_types=[
        pltpu.VMEM((C,), jnp.int32),
        pltpu.VMEM((C,), jnp.int32),
        pltpu.VMEM((C,), jnp.int32),
        pltpu.VMEM((C,), jnp.int32),
        pltpu.VMEM((C, D), jnp.float32),
        pltpu.VMEM((C, D), jnp.float32),
        pltpu.VMEM((C,), jnp.float32),
        pltpu.VMEM((TILE_ROWS,), jnp.float32),
        pltpu.VMEM_SHARED((NP, D), jnp.float32),
        pltpu.VMEM_SHARED((NP,), jnp.float32),
        pltpu.SemaphoreType.DMA,
        pltpu.SemaphoreType.DMA,
        pltpu.SemaphoreType.DMA,
        pltpu.SemaphoreType.DMA,
    ],
)
def _sc_seg_sum(x_hbm, src_hbm, dst_hbm, sums_hbm, cnts_hbm,
                srcv0, dstv0, srcv1, dstv1, rows, rows1, ones, zcnt,
                acc_sh, cnt_sh, sem, sem1, cs0, cs1):
    cid = lax.axis_index("c")
    sid = lax.axis_index("s")
    srcv = (srcv0, srcv1)
    dstv = (dstv0, dstv1)
    rowsb = (rows, rows1)
    gsem = (sem, sem1)
    csem = (cs0, cs1)

    @pl.loop(0, C, step=16)
    def _(j):
        ones[pl.ds(j, 16)] = jnp.ones((16,), jnp.float32)

    @pl.loop(0, C)
    def _(i):
        @pl.loop(0, D, step=16)
        def _(j):
            rows[i, pl.ds(j, 16)] = jnp.zeros((16,), jnp.float32)

    @pl.loop(0, TILE_ROWS, step=16)
    def _(j):
        zcnt[pl.ds(j, 16)] = jnp.zeros((16,), jnp.float32)

    base = sid * TILE_ROWS
    for k in range(TILE_ROWS // C):
        pltpu.sync_copy(rows, acc_sh.at[pl.ds(base + k * C, C)])
    pltpu.sync_copy(zcnt, cnt_sh.at[pl.ds(base, TILE_ROWS)])
    plsc.subcore_barrier()

    lo = cid * (R // 2) + sid
    hi = (cid + 1) * (R // 2)

    pltpu.sync_copy(src_hbm.at[lo], srcv0)
    pltpu.sync_copy(dst_hbm.at[lo], dstv0)
    pltpu.async_copy(x_hbm.at[srcv0], rows, sem)

    @pl.loop(lo, hi, step=32)
    def _(r0):
        for b in range(2):
            r = r0 + b * 16
            p, q = b, 1 - b

            @pl.when(r < hi)
            def _():
                @pl.when(r + 16 < hi)
                def _():
                    @pl.when(r > lo)
                    def _():
                        pltpu.make_async_copy(ones,
                                              cnt_sh.at[pl.ds(0, C)],
                                              csem[q]).wait()
                    pltpu.sync_copy(src_hbm.at[r + 16], srcv[q])
                    pltpu.sync_copy(dst_hbm.at[r + 16], dstv[q])
                    pltpu.async_copy(x_hbm.at[srcv[q]], rowsb[q],
                                     gsem[q])

                pltpu.make_async_copy(x_hbm.at[pl.ds(0, C)], rowsb[p],
                                      gsem[p]).wait()
                pltpu.sync_copy(rowsb[p], acc_sh.at[dstv[p]], add=True)
                pltpu.async_copy(ones, cnt_sh.at[dstv[p]], csem[p],
                                 add=True)

    for p in range(2):
        pltpu.make_async_copy(ones, cnt_sh.at[pl.ds(0, C)],
                              csem[p]).wait()

    plsc.subcore_barrier()

    pltpu.sync_copy(acc_sh.at[pl.ds(base, TILE_ROWS)],
                    sums_hbm.at[cid, pl.ds(base, TILE_ROWS)])
    pltpu.sync_copy(cnt_sh.at[pl.ds(base, TILE_ROWS)],
                    cnts_hbm.at[cid, pl.ds(base, TILE_ROWS)])


@functools.partial(
    pl.kernel,
    out_type=jax.ShapeDtypeStruct((2, NP), jnp.float32),
    mesh=_mesh,
    compiler_params=_sc_params,
    scratch_types=[
        pltpu.VMEM((KPT, C), jnp.int32),
        pltpu.VMEM((KPT, C), jnp.int32),
        pltpu.VMEM((KPT, C), jnp.float32),
        pltpu.VMEM((NP,), jnp.float32),
        pltpu.VMEM((TILE_ROWS,), jnp.float32),
        pltpu.VMEM_SHARED((NP,), jnp.float32),
        pltpu.SemaphoreType.DMA,
    ],
)
def _sc_seg_sum_scalar(s2_hbm, src_hbm, dst_hbm, parts_hbm,
                       srcslab, dstslab, vals, s2loc, zcnt, acc_sh, ssem):
    cid = lax.axis_index("c")
    sid = lax.axis_index("s")
    wid = sid * 2 + cid
    base = sid * TILE_ROWS

    pltpu.sync_copy(src_hbm.at[pl.ds(wid * KPT, KPT)], srcslab)
    pltpu.sync_copy(dst_hbm.at[pl.ds(wid * KPT, KPT)], dstslab)
    pltpu.sync_copy(s2_hbm, s2loc)

    @pl.loop(0, TILE_ROWS, step=16)
    def _(j):
        zcnt[pl.ds(j, 16)] = jnp.zeros((16,), jnp.float32)

    pltpu.sync_copy(zcnt, acc_sh.at[pl.ds(base, TILE_ROWS)])
    plsc.subcore_barrier()

    @pl.loop(0, KPT)
    def _(k):
        for j in range(C // 16):
            idx = srcslab[k, pl.ds(j * 16, 16)]
            vals[k, pl.ds(j * 16, 16)] = plsc.load_gather(s2loc, [idx])
        pltpu.async_copy(vals.at[k], acc_sh.at[dstslab.at[k]], ssem,
                         add=True)

    pltpu.make_async_copy(s2_hbm, s2loc, ssem).wait()
    plsc.subcore_barrier()
    pltpu.sync_copy(acc_sh.at[pl.ds(base, TILE_ROWS)],
                    parts_hbm.at[cid, pl.ds(base, TILE_ROWS)])


def _tc_layer1(s_ref, c_ref, x_ref, wl1_ref, bl1_ref, wr1_ref,
               wl2_ref, wr2_ref, bl2_ref, s2_ref, r2b_ref, cntc_ref):
    seg = s_ref[0] + s_ref[1]
    cnt = jnp.maximum(c_ref[0] + c_ref[1], 1.0)
    agg = seg / cnt
    h = agg @ wl1_ref[...] + bl1_ref[...] + x_ref[...] @ wr1_ref[...]
    h = jnp.maximum(h, 0.0)
    s2_ref[...] = h @ wl2_ref[...]
    r2b_ref[...] = h @ wr2_ref[...] + bl2_ref[...]
    cntc_ref[...] = cnt


def _tc_head(p_ref, cnt_ref, r2b_ref, noise_ref, wmu_ref, bmu_ref,
             wlv_ref, blv_ref, z_ref):
    xm = (p_ref[0] + p_ref[1]) / cnt_ref[...] + r2b_ref[...]
    xm = xm[:N]
    mu = xm * wmu_ref[0, 0] + bmu_ref[0, 0]
    lv = xm * wlv_ref[0, 0] + blv_ref[0, 0]
    z_ref[...] = mu + noise_ref[...] * jnp.exp(lv)


def kernel(x, edge_index, W_l1, b_l1, W_r1, W_l2, b_l2, W_r2,
           w_mu, b_mu, w_lv, b_lv, noise):
    src2 = edge_index[0].reshape(R, C)
    dst2 = edge_index[1].reshape(R, C)
    srcp = jnp.concatenate(
        [edge_index[0], jnp.zeros((EP - E,), jnp.int32)]).reshape(RP, C)
    dstp = jnp.concatenate(
        [edge_index[1], jnp.full((EP - E,), NP - 1, jnp.int32)]
    ).reshape(RP, C)
    xp = jnp.pad(x, ((0, NP - N), (0, 0)))

    sums, cnts = _sc_seg_sum(xp, src2, dst2)

    B = 640
    s2, r2b, cntc = pl.pallas_call(
        _tc_layer1,
        grid=(NP // B,),
        in_specs=[
            pl.BlockSpec((2, B, D), lambda i: (0, i, 0)),
            pl.BlockSpec((2, B, 1), lambda i: (0, i, 0)),
            pl.BlockSpec((B, D), lambda i: (i, 0)),
            pl.BlockSpec((D, H), lambda i: (0, 0)),
            pl.BlockSpec((1, H), lambda i: (0, 0)),
            pl.BlockSpec((D, H), lambda i: (0, 0)),
            pl.BlockSpec((H, 1), lambda i: (0, 0)),
            pl.BlockSpec((H, 1), lambda i: (0, 0)),
            pl.BlockSpec((1, 1), lambda i: (0, 0)),
        ],
        out_specs=[
            pl.BlockSpec((B, 1), lambda i: (i, 0)),
            pl.BlockSpec((B, 1), lambda i: (i, 0)),
            pl.BlockSpec((B, 1), lambda i: (i, 0)),
        ],
        out_shape=[
            jax.ShapeDtypeStruct((NP, 1), jnp.float32),
            jax.ShapeDtypeStruct((NP, 1), jnp.float32),
            jax.ShapeDtypeStruct((NP, 1), jnp.float32),
        ],
    )(sums, cnts.reshape(2, NP, 1), xp, W_l1, b_l1.reshape(1, H), W_r1,
      W_l2, W_r2, b_l2.reshape(1, 1))

    parts2 = _sc_seg_sum_scalar(s2.reshape(NP), srcp, dstp)

    z = pl.pallas_call(
        _tc_head,
        grid=(1,),
        in_specs=[
            pl.BlockSpec((2, NP, 1), lambda i: (0, 0, 0)),
            pl.BlockSpec((NP, 1), lambda i: (0, 0)),
            pl.BlockSpec((NP, 1), lambda i: (0, 0)),
            pl.BlockSpec((N, 1), lambda i: (0, 0)),
            pl.BlockSpec((1, 1), lambda i: (0, 0)),
            pl.BlockSpec((1, 1), lambda i: (0, 0)),
            pl.BlockSpec((1, 1), lambda i: (0, 0)),
            pl.BlockSpec((1, 1), lambda i: (0, 0)),
        ],
        out_specs=pl.BlockSpec((N, 1), lambda i: (0, 0)),
        out_shape=jax.ShapeDtypeStruct((N, 1), jnp.float32),
    )(parts2.reshape(2, NP, 1), cntc, r2b, noise,
      w_mu, b_mu.reshape(1, 1), w_lv, b_lv.reshape(1, 1))

    return z


# speedup vs baseline: 1.7240x; 1.1914x over previous
import dataclasses
import functools

import jax
import jax.numpy as jnp
from jax import lax
from jax.experimental import pallas as pl
from jax.experimental.pallas import tpu as pltpu
from jax.experimental.pallas import tpu_sc as plsc

N = 10000
E = 320000
D = 128
H = 128

C = 128
R = E // C
KPT = 80
RP = 32 * KPT
EP = RP * C
NP = 10240
TILE_ROWS = NP // 16

_mesh = plsc.VectorSubcoreMesh(core_axis_name="c", subcore_axis_name="s")

_sc_params = pltpu.CompilerParams()
if "needs_layout_passes" in pltpu.CompilerParams.__dataclass_fields__:
    _sc_params = dataclasses.replace(_sc_params, needs_layout_passes=False)


@functools.partial(
    pl.kernel,
    out_type=[
        jax.ShapeDtypeStruct((2, NP, D), jnp.float32),
        jax.ShapeDtypeStruct((2, NP), jnp.float32),
    ],
    mesh=_mesh,
    scratch_types=[
        pltpu.VMEM((2, C), jnp.int32),
        pltpu.VMEM((2, C), jnp.int32),
        pltpu.VMEM((C, D), jnp.float32),
        pltpu.VMEM((C, D), jnp.float32),
        pltpu.VMEM((C,), jnp.float32),
        pltpu.VMEM((TILE_ROWS,), jnp.float32),
        pltpu.VMEM_SHARED((NP, D), jnp.float32),
        pltpu.VMEM_SHARED((NP,), jnp.float32),
        pltpu.SemaphoreType.DMA,
        pltpu.SemaphoreType.DMA,
        pltpu.SemaphoreType.DMA,
        pltpu.SemaphoreType.DMA,
    ],
)
def _sc_seg_sum(x_hbm, ei_hbm, sums_hbm, cnts_hbm,
                idx0, idx1, rows, rows1, ones, zcnt,
                acc_sh, cnt_sh, sem, sem1, cs0, cs1):
    cid = lax.axis_index("c")
    sid = lax.axis_index("s")
    idxb = (idx0, idx1)
    rowsb = (rows, rows1)
    gsem = (sem, sem1)
    csem = (cs0, cs1)

    @pl.loop(0, C, step=16)
    def _(j):
        ones[pl.ds(j, 16)] = jnp.ones((16,), jnp.float32)

    @pl.loop(0, C)
    def _(i):
        @pl.loop(0, D, step=16)
        def _(j):
            rows[i, pl.ds(j, 16)] = jnp.zeros((16,), jnp.float32)

    @pl.loop(0, TILE_ROWS, step=16)
    def _(j):
        zcnt[pl.ds(j, 16)] = jnp.zeros((16,), jnp.float32)

    base = sid * TILE_ROWS
    for k in range(TILE_ROWS // C):
        pltpu.sync_copy(rows, acc_sh.at[pl.ds(base + k * C, C)])
    pltpu.sync_copy(zcnt, cnt_sh.at[pl.ds(base, TILE_ROWS)])
    plsc.subcore_barrier()

    lo = cid * (R // 2) + sid
    hi = (cid + 1) * (R // 2)

    pltpu.sync_copy(ei_hbm.at[lo], idx0)
    pltpu.async_copy(x_hbm.at[idx0.at[0]], rows, sem)

    @pl.loop(lo, hi, step=32)
    def _(r0):
        for b in range(2):
            r = r0 + b * 16
            p, q = b, 1 - b

            @pl.when(r < hi)
            def _():
                @pl.when(r + 16 < hi)
                def _():
                    @pl.when(r > lo)
                    def _():
                        pltpu.make_async_copy(ones,
                                              cnt_sh.at[pl.ds(0, C)],
                                              csem[q]).wait()
                    pltpu.sync_copy(ei_hbm.at[r + 16], idxb[q])
                    pltpu.async_copy(x_hbm.at[idxb[q].at[0]], rowsb[q],
                                     gsem[q])

                pltpu.make_async_copy(x_hbm.at[pl.ds(0, C)], rowsb[p],
                                      gsem[p]).wait()
                pltpu.sync_copy(rowsb[p], acc_sh.at[idxb[p].at[1]],
                                add=True)
                pltpu.async_copy(ones, cnt_sh.at[idxb[p].at[1]],
                                 csem[p], add=True)

    for p in range(2):
        pltpu.make_async_copy(ones, cnt_sh.at[pl.ds(0, C)],
                              csem[p]).wait()

    plsc.subcore_barrier()

    pltpu.sync_copy(acc_sh.at[pl.ds(base, TILE_ROWS)],
                    sums_hbm.at[cid, pl.ds(base, TILE_ROWS)])
    pltpu.sync_copy(cnt_sh.at[pl.ds(base, TILE_ROWS)],
                    cnts_hbm.at[cid, pl.ds(base, TILE_ROWS)])


@functools.partial(
    pl.kernel,
    out_type=jax.ShapeDtypeStruct((2, NP), jnp.float32),
    mesh=_mesh,
    compiler_params=_sc_params,
    scratch_types=[
        pltpu.VMEM((KPT, C), jnp.int32),
        pltpu.VMEM((KPT, C), jnp.int32),
        pltpu.VMEM((KPT, C), jnp.float32),
        pltpu.VMEM((NP,), jnp.float32),
        pltpu.VMEM((TILE_ROWS,), jnp.float32),
        pltpu.VMEM_SHARED((NP,), jnp.float32),
        pltpu.SemaphoreType.DMA,
    ],
)
def _sc_seg_sum_scalar(s2_hbm, src_hbm, dst_hbm, parts_hbm,
                       srcslab, dstslab, vals, s2loc, zcnt, acc_sh, ssem):
    cid = lax.axis_index("c")
    sid = lax.axis_index("s")
    wid = sid * 2 + cid
    base = sid * TILE_ROWS

    pltpu.sync_copy(src_hbm.at[pl.ds(wid * KPT, KPT)], srcslab)
    pltpu.sync_copy(dst_hbm.at[pl.ds(wid * KPT, KPT)], dstslab)
    pltpu.sync_copy(s2_hbm, s2loc)

    @pl.loop(0, TILE_ROWS, step=16)
    def _(j):
        zcnt[pl.ds(j, 16)] = jnp.zeros((16,), jnp.float32)

    pltpu.sync_copy(zcnt, acc_sh.at[pl.ds(base, TILE_ROWS)])
    plsc.subcore_barrier()

    @pl.loop(0, KPT)
    def _(k):
        for j in range(C // 16):
            idx = srcslab[k, pl.ds(j * 16, 16)]
            vals[k, pl.ds(j * 16, 16)] = plsc.load_gather(s2loc, [idx])
        pltpu.async_copy(vals.at[k], acc_sh.at[dstslab.at[k]], ssem,
                         add=True)

    pltpu.make_async_copy(s2_hbm, s2loc, ssem).wait()
    plsc.subcore_barrier()
    pltpu.sync_copy(acc_sh.at[pl.ds(base, TILE_ROWS)],
                    parts_hbm.at[cid, pl.ds(base, TILE_ROWS)])


def _tc_self(x_ref, wr1_ref, bl1_ref, xr_ref):
    xr_ref[...] = x_ref[...] @ wr1_ref[...] + bl1_ref[...]


def _tc_layer1(s_ref, c_ref, xr_ref, wl1_ref,
               wl2_ref, wr2_ref, bl2_ref, s2_ref, r2b_ref, cntc_ref):
    seg = s_ref[0] + s_ref[1]
    cnt = jnp.maximum(c_ref[0] + c_ref[1], 1.0)
    agg = seg / cnt
    h = jnp.maximum(agg @ wl1_ref[...] + xr_ref[...], 0.0)
    s2_ref[...] = h @ wl2_ref[...]
    r2b_ref[...] = h @ wr2_ref[...] + bl2_ref[...]
    cntc_ref[...] = cnt


def _tc_head(p_ref, cnt_ref, r2b_ref, noise_ref, wmu_ref, bmu_ref,
             wlv_ref, blv_ref, z_ref):
    xm = (p_ref[0] + p_ref[1]) / cnt_ref[...] + r2b_ref[...]
    xm = xm[:N]
    mu = xm * wmu_ref[0, 0] + bmu_ref[0, 0]
    lv = xm * wlv_ref[0, 0] + blv_ref[0, 0]
    z_ref[...] = mu + noise_ref[...] * jnp.exp(lv)


def kernel(x, edge_index, W_l1, b_l1, W_r1, W_l2, b_l2, W_r2,
           w_mu, b_mu, w_lv, b_lv, noise):
    ei_t = edge_index.reshape(2, R, C).transpose(1, 0, 2)
    srcp = jnp.concatenate(
        [edge_index[0], jnp.zeros((EP - E,), jnp.int32)]).reshape(RP, C)
    dstp = jnp.concatenate(
        [edge_index[1], jnp.full((EP - E,), NP - 1, jnp.int32)]
    ).reshape(RP, C)
    xp = jnp.pad(x, ((0, NP - N), (0, 0)))

    sums, cnts = _sc_seg_sum(x, ei_t)

    B = 640
    xr = pl.pallas_call(
        _tc_self,
        grid=(NP // B,),
        in_specs=[
            pl.BlockSpec((B, D), lambda i: (i, 0)),
            pl.BlockSpec((D, H), lambda i: (0, 0)),
            pl.BlockSpec((1, H), lambda i: (0, 0)),
        ],
        out_specs=pl.BlockSpec((B, H), lambda i: (i, 0)),
        out_shape=jax.ShapeDtypeStruct((NP, H), jnp.float32),
    )(xp, W_r1, b_l1.reshape(1, H))

    s2, r2b, cntc = pl.pallas_call(
        _tc_layer1,
        grid=(NP // B,),
        in_specs=[
            pl.BlockSpec((2, B, D), lambda i: (0, i, 0)),
            pl.BlockSpec((2, B, 1), lambda i: (0, i, 0)),
            pl.BlockSpec((B, H), lambda i: (i, 0)),
            pl.BlockSpec((D, H), lambda i: (0, 0)),
            pl.BlockSpec((H, 1), lambda i: (0, 0)),
            pl.BlockSpec((H, 1), lambda i: (0, 0)),
            pl.BlockSpec((1, 1), lambda i: (0, 0)),
        ],
        out_specs=[
            pl.BlockSpec((B, 1), lambda i: (i, 0)),
            pl.BlockSpec((B, 1), lambda i: (i, 0)),
            pl.BlockSpec((B, 1), lambda i: (i, 0)),
        ],
        out_shape=[
            jax.ShapeDtypeStruct((NP, 1), jnp.float32),
            jax.ShapeDtypeStruct((NP, 1), jnp.float32),
            jax.ShapeDtypeStruct((NP, 1), jnp.float32),
        ],
    )(sums, cnts.reshape(2, NP, 1), xr, W_l1,
      W_l2, W_r2, b_l2.reshape(1, 1))

    parts2 = _sc_seg_sum_scalar(s2.reshape(NP), srcp, dstp)

    z = pl.pallas_call(
        _tc_head,
        grid=(1,),
        in_specs=[
            pl.BlockSpec((2, NP, 1), lambda i: (0, 0, 0)),
            pl.BlockSpec((NP, 1), lambda i: (0, 0)),
            pl.BlockSpec((NP, 1), lambda i: (0, 0)),
            pl.BlockSpec((N, 1), lambda i: (0, 0)),
            pl.BlockSpec((1, 1), lambda i: (0, 0)),
            pl.BlockSpec((1, 1), lambda i: (0, 0)),
            pl.BlockSpec((1, 1), lambda i: (0, 0)),
            pl.BlockSpec((1, 1), lambda i: (0, 0)),
        ],
        out_specs=pl.BlockSpec((N, 1), lambda i: (0, 0)),
        out_shape=jax.ShapeDtypeStruct((N, 1), jnp.float32),
    )(parts2.reshape(2, NP, 1), cntc, r2b, noise,
      w_mu, b_mu.reshape(1, 1), w_lv, b_lv.reshape(1, 1))

    return z


# speedup vs baseline: 1.7370x; 1.0075x over previous
import dataclasses
import functools

import jax
import jax.numpy as jnp
from jax import lax
from jax.experimental import pallas as pl
from jax.experimental.pallas import tpu as pltpu
from jax.experimental.pallas import tpu_sc as plsc

N = 10000
E = 320000
D = 128
H = 128

C = 128
R = E // C
KPT = 80
RP = 32 * KPT
EP = RP * C
NP = 10240
TILE_ROWS = NP // 16

_mesh = plsc.VectorSubcoreMesh(core_axis_name="c", subcore_axis_name="s")

_sc_params = pltpu.CompilerParams()
if "needs_layout_passes" in pltpu.CompilerParams.__dataclass_fields__:
    _sc_params = dataclasses.replace(_sc_params, needs_layout_passes=False)


@functools.partial(
    pl.kernel,
    out_type=[
        jax.ShapeDtypeStruct((2, NP, D), jnp.float32),
        jax.ShapeDtypeStruct((2, NP), jnp.float32),
    ],
    mesh=_mesh,
    scratch_types=[
        pltpu.VMEM((2, C), jnp.int32),
        pltpu.VMEM((2, C), jnp.int32),
        pltpu.VMEM((C, D), jnp.float32),
        pltpu.VMEM((C, D), jnp.float32),
        pltpu.VMEM((C,), jnp.float32),
        pltpu.VMEM((TILE_ROWS,), jnp.float32),
        pltpu.VMEM_SHARED((NP, D), jnp.float32),
        pltpu.VMEM_SHARED((NP,), jnp.float32),
        pltpu.SemaphoreType.DMA,
        pltpu.SemaphoreType.DMA,
        pltpu.SemaphoreType.DMA,
        pltpu.SemaphoreType.DMA,
        pltpu.SemaphoreType.DMA,
    ],
)
def _sc_seg_sum(x_hbm, ei_hbm, sums_hbm, cnts_hbm,
                idx0, idx1, rows, rows1, ones, zcnt,
                acc_sh, cnt_sh, sem, sem1, is0, is1, csem):
    cid = lax.axis_index("c")
    sid = lax.axis_index("s")
    idxb = (idx0, idx1)
    rowsb = (rows, rows1)
    gsem = (sem, sem1)
    isem = (is0, is1)

    @pl.loop(0, C, step=16)
    def _(j):
        ones[pl.ds(j, 16)] = jnp.ones((16,), jnp.float32)

    @pl.loop(0, C)
    def _(i):
        @pl.loop(0, D, step=16)
        def _(j):
            rows[i, pl.ds(j, 16)] = jnp.zeros((16,), jnp.float32)

    @pl.loop(0, TILE_ROWS, step=16)
    def _(j):
        zcnt[pl.ds(j, 16)] = jnp.zeros((16,), jnp.float32)

    base = sid * TILE_ROWS
    for k in range(TILE_ROWS // C):
        pltpu.sync_copy(rows, acc_sh.at[pl.ds(base + k * C, C)])
    pltpu.sync_copy(zcnt, cnt_sh.at[pl.ds(base, TILE_ROWS)])
    plsc.subcore_barrier()

    lo = cid * (R // 2) + sid
    hi = (cid + 1) * (R // 2)

    pltpu.sync_copy(ei_hbm.at[lo], idx0)
    pltpu.async_copy(x_hbm.at[idx0.at[0]], rows, sem)
    pltpu.async_copy(ei_hbm.at[lo + 16], idx1, is1)

    @pl.loop(lo, hi, step=32)
    def _(r0):
        for b in range(2):
            r = r0 + b * 16
            p, q = b, 1 - b

            @pl.when(r < hi)
            def _():
                @pl.when(r + 16 < hi)
                def _():
                    pltpu.make_async_copy(ei_hbm.at[0], idxb[q],
                                          isem[q]).wait()
                    pltpu.async_copy(x_hbm.at[idxb[q].at[0]], rowsb[q],
                                     gsem[q])

                pltpu.make_async_copy(x_hbm.at[pl.ds(0, C)], rowsb[p],
                                      gsem[p]).wait()
                pltpu.async_copy(ones, cnt_sh.at[idxb[p].at[1]],
                                 csem, add=True)
                pltpu.sync_copy(rowsb[p], acc_sh.at[idxb[p].at[1]],
                                add=True)
                pltpu.make_async_copy(ones, cnt_sh.at[pl.ds(0, C)],
                                      csem).wait()

                @pl.when(r + 32 < hi)
                def _():
                    pltpu.async_copy(ei_hbm.at[r + 32], idxb[p],
                                     isem[p])

    plsc.subcore_barrier()

    pltpu.sync_copy(acc_sh.at[pl.ds(base, TILE_ROWS)],
                    sums_hbm.at[cid, pl.ds(base, TILE_ROWS)])
    pltpu.sync_copy(cnt_sh.at[pl.ds(base, TILE_ROWS)],
                    cnts_hbm.at[cid, pl.ds(base, TILE_ROWS)])


@functools.partial(
    pl.kernel,
    out_type=jax.ShapeDtypeStruct((2, NP), jnp.float32),
    mesh=_mesh,
    compiler_params=_sc_params,
    scratch_types=[
        pltpu.VMEM((KPT, C), jnp.int32),
        pltpu.VMEM((KPT, C), jnp.int32),
        pltpu.VMEM((KPT, C), jnp.float32),
        pltpu.VMEM((NP,), jnp.float32),
        pltpu.VMEM((TILE_ROWS,), jnp.float32),
        pltpu.VMEM_SHARED((NP,), jnp.float32),
        pltpu.SemaphoreType.DMA,
    ],
)
def _sc_seg_sum_scalar(s2_hbm, src_hbm, dst_hbm, parts_hbm,
                       srcslab, dstslab, vals, s2loc, zcnt, acc_sh, ssem):
    cid = lax.axis_index("c")
    sid = lax.axis_index("s")
    wid = sid * 2 + cid
    base = sid * TILE_ROWS

    pltpu.sync_copy(src_hbm.at[pl.ds(wid * KPT, KPT)], srcslab)
    pltpu.sync_copy(dst_hbm.at[pl.ds(wid * KPT, KPT)], dstslab)
    pltpu.sync_copy(s2_hbm, s2loc)

    @pl.loop(0, TILE_ROWS, step=16)
    def _(j):
        zcnt[pl.ds(j, 16)] = jnp.zeros((16,), jnp.float32)

    pltpu.sync_copy(zcnt, acc_sh.at[pl.ds(base, TILE_ROWS)])
    plsc.subcore_barrier()

    @pl.loop(0, KPT)
    def _(k):
        for j in range(C // 16):
            idx = srcslab[k, pl.ds(j * 16, 16)]
            vals[k, pl.ds(j * 16, 16)] = plsc.load_gather(s2loc, [idx])
        pltpu.async_copy(vals.at[k], acc_sh.at[dstslab.at[k]], ssem,
                         add=True)

    pltpu.make_async_copy(s2_hbm, s2loc, ssem).wait()
    plsc.subcore_barrier()
    pltpu.sync_copy(acc_sh.at[pl.ds(base, TILE_ROWS)],
                    parts_hbm.at[cid, pl.ds(base, TILE_ROWS)])


def _tc_self(x_ref, wr1_ref, bl1_ref, xr_ref):
    xr_ref[...] = x_ref[...] @ wr1_ref[...] + bl1_ref[...]


def _tc_layer1(s_ref, c_ref, xr_ref, wl1_ref,
               wl2_ref, wr2_ref, bl2_ref, s2_ref, r2b_ref, cntc_ref):
    seg = s_ref[0] + s_ref[1]
    cnt = jnp.maximum(c_ref[0] + c_ref[1], 1.0)
    agg = seg / cnt
    h = jnp.maximum(agg @ wl1_ref[...] + xr_ref[...], 0.0)
    s2_ref[...] = h @ wl2_ref[...]
    r2b_ref[...] = h @ wr2_ref[...] + bl2_ref[...]
    cntc_ref[...] = cnt


def _tc_head(p_ref, cnt_ref, r2b_ref, noise_ref, wmu_ref, bmu_ref,
             wlv_ref, blv_ref, z_ref):
    xm = (p_ref[0] + p_ref[1]) / cnt_ref[...] + r2b_ref[...]
    xm = xm[:N]
    mu = xm * wmu_ref[0, 0] + bmu_ref[0, 0]
    lv = xm * wlv_ref[0, 0] + blv_ref[0, 0]
    z_ref[...] = mu + noise_ref[...] * jnp.exp(lv)


def kernel(x, edge_index, W_l1, b_l1, W_r1, W_l2, b_l2, W_r2,
           w_mu, b_mu, w_lv, b_lv, noise):
    ei_t = edge_index.reshape(2, R, C).transpose(1, 0, 2)
    srcp = jnp.concatenate(
        [edge_index[0], jnp.zeros((EP - E,), jnp.int32)]).reshape(RP, C)
    dstp = jnp.concatenate(
        [edge_index[1], jnp.full((EP - E,), NP - 1, jnp.int32)]
    ).reshape(RP, C)
    xp = jnp.pad(x, ((0, NP - N), (0, 0)))

    sums, cnts = _sc_seg_sum(x, ei_t)

    B = 640
    xr = pl.pallas_call(
        _tc_self,
        grid=(NP // B,),
        in_specs=[
            pl.BlockSpec((B, D), lambda i: (i, 0)),
            pl.BlockSpec((D, H), lambda i: (0, 0)),
            pl.BlockSpec((1, H), lambda i: (0, 0)),
        ],
        out_specs=pl.BlockSpec((B, H), lambda i: (i, 0)),
        out_shape=jax.ShapeDtypeStruct((NP, H), jnp.float32),
    )(xp, W_r1, b_l1.reshape(1, H))

    s2, r2b, cntc = pl.pallas_call(
        _tc_layer1,
        grid=(NP // B,),
        in_specs=[
            pl.BlockSpec((2, B, D), lambda i: (0, i, 0)),
            pl.BlockSpec((2, B, 1), lambda i: (0, i, 0)),
            pl.BlockSpec((B, H), lambda i: (i, 0)),
            pl.BlockSpec((D, H), lambda i: (0, 0)),
            pl.BlockSpec((H, 1), lambda i: (0, 0)),
            pl.BlockSpec((H, 1), lambda i: (0, 0)),
            pl.BlockSpec((1, 1), lambda i: (0, 0)),
        ],
        out_specs=[
            pl.BlockSpec((B, 1), lambda i: (i, 0)),
            pl.BlockSpec((B, 1), lambda i: (i, 0)),
            pl.BlockSpec((B, 1), lambda i: (i, 0)),
        ],
        out_shape=[
            jax.ShapeDtypeStruct((NP, 1), jnp.float32),
            jax.ShapeDtypeStruct((NP, 1), jnp.float32),
            jax.ShapeDtypeStruct((NP, 1), jnp.float32),
        ],
    )(sums, cnts.reshape(2, NP, 1), xr, W_l1,
      W_l2, W_r2, b_l2.reshape(1, 1))

    parts2 = _sc_seg_sum_scalar(s2.reshape(NP), srcp, dstp)

    z = pl.pallas_call(
        _tc_head,
        grid=(1,),
        in_specs=[
            pl.BlockSpec((2, NP, 1), lambda i: (0, 0, 0)),
            pl.BlockSpec((NP, 1), lambda i: (0, 0)),
            pl.BlockSpec((NP, 1), lambda i: (0, 0)),
            pl.BlockSpec((N, 1), lambda i: (0, 0)),
            pl.BlockSpec((1, 1), lambda i: (0, 0)),
            pl.BlockSpec((1, 1), lambda i: (0, 0)),
            pl.BlockSpec((1, 1), lambda i: (0, 0)),
            pl.BlockSpec((1, 1), lambda i: (0, 0)),
        ],
        out_specs=pl.BlockSpec((N, 1), lambda i: (0, 0)),
        out_shape=jax.ShapeDtypeStruct((N, 1), jnp.float32),
    )(parts2.reshape(2, NP, 1), cntc, r2b, noise,
      w_mu, b_mu.reshape(1, 1), w_lv, b_lv.reshape(1, 1))

    return z
